# baseline (speedup 1.0000x reference)
import jax
import jax.numpy as jnp
from jax.experimental import pallas as pl

_G = 8
_NP = 32
_BD = _G * _NP
_NG = 16
_GA = _G * _NG
_K = 10
_N = 30


def _body(real_ref, graph_ref, w1t_ref, b1_ref, w2t_ref, b2_ref,
          wlin_ref, blin_ref, wconv_ref, bconv_ref, out_ref):
    f32 = jnp.float32
    ir = jax.lax.broadcasted_iota(jnp.int32, (_N, _N), 0)
    ic = jax.lax.broadcasted_iota(jnp.int32, (_N, _N), 1)
    eye30 = jnp.where(ir == ic, 1.0, 0.0).astype(f32)
    row = jax.lax.broadcasted_iota(jnp.int32, (_BD, _BD), 0)
    col = jax.lax.broadcasted_iota(jnp.int32, (_BD, _BD), 1)
    blkmask = (row // _NP) == (col // _NP)
    ipad = jnp.where((row == col) & (row % _NP < _N), 1.0, 0.0).astype(f32)
    ipad_bf = ipad.astype(jnp.bfloat16)

    gr = graph_ref[...]
    a9s = []
    for g in range(_NG):
        adj = jnp.sum(gr[g * _G:(g + 1) * _G], axis=1)
        a_hat = (adj != 0).astype(f32) + eye30[None]
        deg = jnp.sum(a_hat, axis=2)
        dinv = jax.lax.rsqrt(deg)
        norm = dinv[:, :, None] * a_hat * dinv[:, None, :]
        normp = jnp.pad(norm, ((0, 0), (0, 2), (0, 2)))
        flat = normp.reshape(_BD, _NP)
        tiled = jnp.concatenate([flat] * _G, axis=1)
        bd = jnp.where(blkmask, tiled, 0.0)
        a9s.append((0.9 * bd.T).astype(jnp.bfloat16))

    xp = jnp.pad(real_ref[...], ((0, 0), (0, 2), (0, 0))).reshape(_NG * _BD, 128)
    h1 = jnp.maximum(
        jnp.dot(xp, w1t_ref[...], preferred_element_type=f32) + b1_ref[...], 0.0)
    z = jnp.dot(h1, w2t_ref[...], preferred_element_type=f32)

    bf16 = jnp.bfloat16

    def _mm(a, b):
        return jnp.dot(a, b, preferred_element_type=f32)

    g2s = [(ipad_bf + a9s[g]).astype(bf16) for g in range(_NG)]
    p2s = [_mm(a9s[g], a9s[g]).astype(bf16) for g in range(_NG)]
    g4s = [(g2s[g] + _mm(p2s[g], g2s[g])).astype(bf16) for g in range(_NG)]
    p4s = [_mm(p2s[g], p2s[g]).astype(bf16) for g in range(_NG)]
    g8s = [(g4s[g] + _mm(p4s[g], g4s[g])).astype(bf16) for g in range(_NG)]
    p8s = [_mm(p4s[g], p4s[g]).astype(bf16) for g in range(_NG)]
    g10s = [(g8s[g] + _mm(p8s[g], g2s[g])).astype(bf16) for g in range(_NG)]
    p10s = [_mm(p8s[g], p2s[g]) for g in range(_NG)]
    ts = tuple((p10s[g] + 0.1 * g10s[g].astype(f32)).astype(bf16)
               for g in range(_NG))

    ys = [jnp.maximum(
        jnp.dot(ts[g], z[g * _BD:(g + 1) * _BD], preferred_element_type=f32)
        + b2_ref[...], 0.0) for g in range(_NG)]
    y = jnp.concatenate(ys, axis=0)
    v = jnp.dot(y, wlin_ref[...], preferred_element_type=f32)
    us = [jnp.dot(ts[g], v[g * _BD:(g + 1) * _BD], preferred_element_type=f32)
          for g in range(_NG)]
    u = jnp.concatenate(us, axis=0)
    xf = jnp.maximum(u + blin_ref[0, 0], 0.0)
    prod = xf * wconv_ref[...]
    out_ref[0] = jnp.sum(prod.reshape(_GA, _NP, 4), axis=1) + bconv_ref[...]


def kernel(real, imag, graph, layer, W1, b1, W2, b2, Wlin, blin, Wconv, bconv):
    del imag, layer
    B = real.shape[0]
    w1t = W1.T
    w2t = W2.T
    b1r = b1.reshape(1, 128)
    b2r = b2.reshape(1, 128)
    wlin_c = Wlin.reshape(128, 1)
    blin_r = blin.reshape(1, 1)
    wconv_pad = jnp.pad(Wconv[:, :, 0].T, ((0, 2), (0, 0)))
    wconv_big = jnp.tile(wconv_pad, (_GA, 1))
    bconv_r = bconv.reshape(1, 4)
    grid = (B // _GA,)
    return pl.pallas_call(
        _body,
        grid=grid,
        in_specs=[
            pl.BlockSpec((_GA, _N, 128), lambda i: (i, 0, 0)),
            pl.BlockSpec((_GA, 4, _N, _N), lambda i: (i, 0, 0, 0)),
            pl.BlockSpec((128, 128), lambda i: (0, 0)),
            pl.BlockSpec((1, 128), lambda i: (0, 0)),
            pl.BlockSpec((128, 128), lambda i: (0, 0)),
            pl.BlockSpec((1, 128), lambda i: (0, 0)),
            pl.BlockSpec((128, 1), lambda i: (0, 0)),
            pl.BlockSpec((1, 1), lambda i: (0, 0)),
            pl.BlockSpec((_GA * _NP, 4), lambda i: (0, 0)),
            pl.BlockSpec((1, 4), lambda i: (0, 0)),
        ],
        out_specs=pl.BlockSpec((1, _GA, 4), lambda i: (i, 0, 0)),
        out_shape=jax.ShapeDtypeStruct((B // _GA, _GA, 4), jnp.float32),
    )(real, graph, w1t, b1r, w2t, b2r, wlin_c, blin_r, wconv_big, bconv_r
      ).reshape(B, 4)


# speedup vs baseline: 1.0070x; 1.0070x over previous
import jax
import jax.numpy as jnp
from jax.experimental import pallas as pl

_G = 8
_NP = 32
_BD = _G * _NP
_NG = 8
_GA = _G * _NG
_K = 10
_N = 30


def _body(real_ref, graph_ref, w1t_ref, b1_ref, w2t_ref, b2_ref,
          wlin_ref, blin_ref, wconv_ref, bconv_ref, out_ref):
    f32 = jnp.float32
    ir = jax.lax.broadcasted_iota(jnp.int32, (_N, _N), 0)
    ic = jax.lax.broadcasted_iota(jnp.int32, (_N, _N), 1)
    eye30 = jnp.where(ir == ic, 1.0, 0.0).astype(f32)
    row = jax.lax.broadcasted_iota(jnp.int32, (_BD, _BD), 0)
    col = jax.lax.broadcasted_iota(jnp.int32, (_BD, _BD), 1)
    blkmask = (row // _NP) == (col // _NP)
    ipad = jnp.where((row == col) & (row % _NP < _N), 1.0, 0.0).astype(f32)
    ipad_bf = ipad.astype(jnp.bfloat16)

    gr = graph_ref[...]
    a9s = []
    for g in range(_NG):
        adj = jnp.sum(gr[g * _G:(g + 1) * _G], axis=1)
        a_hat = (adj != 0).astype(f32) + eye30[None]
        deg = jnp.sum(a_hat, axis=2)
        dinv = jax.lax.rsqrt(deg)
        norm = dinv[:, :, None] * a_hat * dinv[:, None, :]
        normp = jnp.pad(norm, ((0, 0), (0, 2), (0, 2)))
        flat = normp.reshape(_BD, _NP)
        tiled = jnp.concatenate([flat] * _G, axis=1)
        bd = jnp.where(blkmask, tiled, 0.0)
        a9s.append((0.9 * bd.T).astype(jnp.bfloat16))

    xp = jnp.pad(real_ref[...], ((0, 0), (0, 2), (0, 0))).reshape(_NG * _BD, 128)
    h1 = jnp.maximum(
        jnp.dot(xp, w1t_ref[...], preferred_element_type=f32) + b1_ref[...], 0.0)
    z = jnp.dot(h1, w2t_ref[...], preferred_element_type=f32)

    bf16 = jnp.bfloat16

    def _mm(a, b):
        return jnp.dot(a, b, preferred_element_type=f32)

    g2s = [(ipad_bf + a9s[g]).astype(bf16) for g in range(_NG)]
    p2s = [_mm(a9s[g], a9s[g]).astype(bf16) for g in range(_NG)]
    g4s = [(g2s[g] + _mm(p2s[g], g2s[g])).astype(bf16) for g in range(_NG)]
    p4s = [_mm(p2s[g], p2s[g]).astype(bf16) for g in range(_NG)]
    g8s = [(g4s[g] + _mm(p4s[g], g4s[g])).astype(bf16) for g in range(_NG)]
    p8s = [_mm(p4s[g], p4s[g]).astype(bf16) for g in range(_NG)]
    g10s = [(g8s[g] + _mm(p8s[g], g2s[g])).astype(bf16) for g in range(_NG)]
    p10s = [_mm(p8s[g], p2s[g]) for g in range(_NG)]
    ts = tuple((p10s[g] + 0.1 * g10s[g].astype(f32)).astype(bf16)
               for g in range(_NG))

    ys = [jnp.maximum(
        jnp.dot(ts[g], z[g * _BD:(g + 1) * _BD], preferred_element_type=f32)
        + b2_ref[...], 0.0) for g in range(_NG)]
    y = jnp.concatenate(ys, axis=0)
    v = jnp.dot(y, wlin_ref[...], preferred_element_type=f32)
    us = [jnp.dot(ts[g], v[g * _BD:(g + 1) * _BD], preferred_element_type=f32)
          for g in range(_NG)]
    u = jnp.concatenate(us, axis=0)
    xf = jnp.maximum(u + blin_ref[0, 0], 0.0)
    prod = xf * wconv_ref[...]
    out_ref[0] = jnp.sum(prod.reshape(_GA, _NP, 4), axis=1) + bconv_ref[...]


def kernel(real, imag, graph, layer, W1, b1, W2, b2, Wlin, blin, Wconv, bconv):
    del imag, layer
    B = real.shape[0]
    w1t = W1.T
    w2t = W2.T
    b1r = b1.reshape(1, 128)
    b2r = b2.reshape(1, 128)
    wlin_c = Wlin.reshape(128, 1)
    blin_r = blin.reshape(1, 1)
    wconv_pad = jnp.pad(Wconv[:, :, 0].T, ((0, 2), (0, 0)))
    wconv_big = jnp.tile(wconv_pad, (_GA, 1))
    bconv_r = bconv.reshape(1, 4)
    grid = (B // _GA,)
    return pl.pallas_call(
        _body,
        grid=grid,
        in_specs=[
            pl.BlockSpec((_GA, _N, 128), lambda i: (i, 0, 0)),
            pl.BlockSpec((_GA, 4, _N, _N), lambda i: (i, 0, 0, 0)),
            pl.BlockSpec((128, 128), lambda i: (0, 0)),
            pl.BlockSpec((1, 128), lambda i: (0, 0)),
            pl.BlockSpec((128, 128), lambda i: (0, 0)),
            pl.BlockSpec((1, 128), lambda i: (0, 0)),
            pl.BlockSpec((128, 1), lambda i: (0, 0)),
            pl.BlockSpec((1, 1), lambda i: (0, 0)),
            pl.BlockSpec((_GA * _NP, 4), lambda i: (0, 0)),
            pl.BlockSpec((1, 4), lambda i: (0, 0)),
        ],
        out_specs=pl.BlockSpec((1, _GA, 4), lambda i: (i, 0, 0)),
        out_shape=jax.ShapeDtypeStruct((B // _GA, _GA, 4), jnp.float32),
    )(real, graph, w1t, b1r, w2t, b2r, wlin_c, blin_r, wconv_big, bconv_r
      ).reshape(B, 4)
